# baseline (speedup 1.0000x reference)
import functools

import numpy as np
import jax
import jax.numpy as jnp
from jax import lax
from jax.experimental import pallas as pl
from jax.experimental.pallas import tpu as pltpu
from jax.experimental.pallas import tpu_sc as plsc

NUM_G = 16
L = 16
NC = 2
NS = 16
NW = NC * NS
B = 2000
D = 8

_MAGIC = np.int32(0x5F3759DF)


def _rsqrt_newton(s2):
    i = lax.bitcast_convert_type(s2, jnp.int32)
    y = lax.bitcast_convert_type(_MAGIC - (i >> 1), jnp.float32)
    for _ in range(3):
        y = y * (np.float32(1.5) - np.float32(0.5) * s2 * y * y)
    return y


def _body(nblk, npad, pos8, ei, shift, scale, out,
          shared, idx_j, idx_i, rows_j, rows_i, obuf, par, sem):
    sid = lax.axis_index("s")
    wid = sid * NC + lax.axis_index("c")
    base = wid * (nblk * B)

    rows_per_tile = npad // NS
    pltpu.sync_copy(pos8.at[pl.ds(sid * rows_per_tile, rows_per_tile)],
                    shared.at[pl.ds(sid * rows_per_tile, rows_per_tile)])
    pltpu.sync_copy(shift, par.at[0])
    pltpu.sync_copy(scale, par.at[1])
    sc = par[1, :]
    par[1, :] = np.float32(-0.5) / (sc * sc)
    plsc.subcore_barrier()

    eidx0 = lax.iota(jnp.int32, L)
    c0 = jnp.zeros((L,), jnp.int32)
    c1 = jnp.ones((L,), jnp.int32)
    c2 = jnp.full((L,), 2, jnp.int32)

    def block(b, carry):
        off = base + b * B
        pltpu.sync_copy(ei.at[0, pl.ds(off, B)], idx_j)
        pltpu.sync_copy(ei.at[1, pl.ds(off, B)], idx_i)
        pltpu.async_copy(shared.at[idx_j], rows_j, sem).wait()
        pltpu.async_copy(shared.at[idx_i], rows_i, sem).wait()

        shift_v = par[0, :]
        neg_inv = par[1, :]

        def grp(k, carry2):
            e0 = k * L
            eidx = e0 + eidx0
            xj = plsc.load_gather(rows_j, [eidx, c0])
            yj = plsc.load_gather(rows_j, [eidx, c1])
            zj = plsc.load_gather(rows_j, [eidx, c2])
            xi = plsc.load_gather(rows_i, [eidx, c0])
            yi = plsc.load_gather(rows_i, [eidx, c1])
            zi = plsc.load_gather(rows_i, [eidx, c2])
            dx = xj - xi
            dy = yj - yi
            dz = zj - zi
            s2 = dx * dx + dy * dy + dz * dz
            n = s2 * _rsqrt_newton(s2)
            n = jnp.where(s2 > np.float32(0.0), n, np.float32(0.0))
            for g in range(NUM_G):
                t = n - shift_v[g]
                obuf[g, pl.ds(e0, L)] = jnp.exp(t * t * neg_inv[g])
            return carry2

        lax.fori_loop(0, B // L, grp, 0, unroll=False)
        pltpu.sync_copy(obuf, out.at[:, pl.ds(off, B)])
        return carry

    lax.fori_loop(0, nblk, block, 0, unroll=False)


def kernel(pos, edge_index, shift, scale):
    n_nodes = pos.shape[0]
    n_edges = edge_index.shape[1]
    ei = edge_index.astype(jnp.int32)
    npad = -(-n_nodes // NS) * NS
    pos8 = jnp.pad(pos.astype(jnp.float32),
                   ((0, npad - n_nodes), (0, D - pos.shape[1])))

    chunk = NW * B
    nblk = -(-n_edges // chunk)
    e_pad = nblk * chunk
    if e_pad != n_edges:
        ei = jnp.pad(ei, ((0, 0), (0, e_pad - n_edges)))

    mesh = plsc.VectorSubcoreMesh(core_axis_name="c", subcore_axis_name="s")
    f = pl.kernel(
        functools.partial(_body, nblk, npad),
        out_type=jax.ShapeDtypeStruct((NUM_G, e_pad), jnp.float32),
        mesh=mesh,
        scratch_types=[
            pltpu.VMEM_SHARED((npad, D), jnp.float32),
            pltpu.VMEM((B,), jnp.int32),
            pltpu.VMEM((B,), jnp.int32),
            pltpu.VMEM((B, D), jnp.float32),
            pltpu.VMEM((B, D), jnp.float32),
            pltpu.VMEM((NUM_G, B), jnp.float32),
            pltpu.VMEM((2, NUM_G), jnp.float32),
            pltpu.SemaphoreType.DMA,
        ],
        compiler_params=pltpu.CompilerParams(
            needs_layout_passes=False,
            use_tc_tiling_on_sc=False,
        ),
        name="gaussian_edge_embed_sc",
    )
    out = f(pos8, ei, shift.astype(jnp.float32), scale.astype(jnp.float32))
    if e_pad != n_edges:
        out = out[:, :n_edges]
    return out.T


# speedup vs baseline: 6.0968x; 6.0968x over previous
import functools

import numpy as np
import jax
import jax.numpy as jnp
from jax import lax
from jax.experimental import pallas as pl
from jax.experimental.pallas import tpu as pltpu
from jax.experimental.pallas import tpu_sc as plsc

NUM_G = 16
L = 16
NC = 2
NS = 16
NW = NC * NS
B = 2048
D = 8

_MAGIC = np.int32(0x5F3759DF)


def _rsqrt_newton(s2):
    i = lax.bitcast_convert_type(s2, jnp.int32)
    y = lax.bitcast_convert_type(_MAGIC - (i >> 1), jnp.float32)
    for _ in range(3):
        y = y * (np.float32(1.5) - np.float32(0.5) * s2 * y * y)
    return y


def _body(nblk, npad, pos8, ei, shift, scale, out,
          shared, idx_j, idx_i, rows_j, rows_i, obuf, par, sem):
    sid = lax.axis_index("s")
    wid = sid * NC + lax.axis_index("c")
    base = wid * (nblk * B)

    rows_per_tile = npad // NS
    pltpu.sync_copy(pos8.at[pl.ds(sid * rows_per_tile, rows_per_tile)],
                    shared.at[pl.ds(sid * rows_per_tile, rows_per_tile)])
    pltpu.sync_copy(shift, par.at[0])
    pltpu.sync_copy(scale, par.at[1])
    sc = par[1, :]
    par[1, :] = np.float32(-0.5) / (sc * sc)
    plsc.subcore_barrier()

    eidx0 = lax.iota(jnp.int32, L)
    c0 = jnp.zeros((L,), jnp.int32)
    c1 = jnp.ones((L,), jnp.int32)
    c2 = jnp.full((L,), 2, jnp.int32)

    def block(b, carry):
        off = base + b * B
        pltpu.sync_copy(ei.at[0, pl.ds(off, B)], idx_j)
        pltpu.sync_copy(ei.at[1, pl.ds(off, B)], idx_i)
        pltpu.async_copy(shared.at[idx_j], rows_j, sem).wait()
        pltpu.async_copy(shared.at[idx_i], rows_i, sem).wait()

        shift_v = par[0, :]
        neg_inv = par[1, :]

        def grp(k, carry2):
            e0 = k * L
            eidx = e0 + eidx0
            xj = plsc.load_gather(rows_j, [eidx, c0])
            yj = plsc.load_gather(rows_j, [eidx, c1])
            zj = plsc.load_gather(rows_j, [eidx, c2])
            xi = plsc.load_gather(rows_i, [eidx, c0])
            yi = plsc.load_gather(rows_i, [eidx, c1])
            zi = plsc.load_gather(rows_i, [eidx, c2])
            dx = xj - xi
            dy = yj - yi
            dz = zj - zi
            s2 = dx * dx + dy * dy + dz * dz
            n = s2 * _rsqrt_newton(s2)
            n = jnp.where(s2 > np.float32(0.0), n, np.float32(0.0))
            cl = k // (128 // L)
            l0 = (k % (128 // L)) * L
            for g in range(NUM_G):
                t = n - shift_v[g]
                obuf[g // 8, cl, g % 8, pl.ds(l0, L)] = jnp.exp(
                    t * t * neg_inv[g])
            return carry2

        lax.fori_loop(0, B // L, grp, 0, unroll=False)
        pltpu.sync_copy(obuf, out.at[:, pl.ds(off // 128, B // 128)])
        return carry

    lax.fori_loop(0, nblk, block, 0, unroll=False)


def kernel(pos, edge_index, shift, scale):
    n_nodes = pos.shape[0]
    n_edges = edge_index.shape[1]
    ei = edge_index.astype(jnp.int32)
    npad = -(-n_nodes // NS) * NS
    pos8 = jnp.pad(pos.astype(jnp.float32),
                   ((0, npad - n_nodes), (0, D - pos.shape[1])))

    chunk = NW * B
    nblk = -(-n_edges // chunk)
    e_pad = nblk * chunk
    if e_pad != n_edges:
        ei = jnp.pad(ei, ((0, 0), (0, e_pad - n_edges)))

    mesh = plsc.VectorSubcoreMesh(core_axis_name="c", subcore_axis_name="s")
    f = pl.kernel(
        functools.partial(_body, nblk, npad),
        out_type=jax.ShapeDtypeStruct((2, e_pad // 128, 8, 128), jnp.float32),
        mesh=mesh,
        scratch_types=[
            pltpu.VMEM_SHARED((npad, D), jnp.float32),
            pltpu.VMEM((B,), jnp.int32),
            pltpu.VMEM((B,), jnp.int32),
            pltpu.VMEM((B, D), jnp.float32),
            pltpu.VMEM((B, D), jnp.float32),
            pltpu.VMEM((2, B // 128, 8, 128), jnp.float32),
            pltpu.VMEM((2, NUM_G), jnp.float32),
            pltpu.SemaphoreType.DMA,
        ],
        compiler_params=pltpu.CompilerParams(
            needs_layout_passes=False,
            use_tc_tiling_on_sc=False,
        ),
        name="gaussian_edge_embed_sc",
    )
    out = f(pos8, ei, shift.astype(jnp.float32), scale.astype(jnp.float32))
    out = out.transpose(1, 3, 0, 2).reshape(e_pad, NUM_G)
    if e_pad != n_edges:
        out = out[:n_edges]
    return out


# speedup vs baseline: 9.0378x; 1.4824x over previous
import functools

import numpy as np
import jax
import jax.numpy as jnp
from jax import lax
from jax.experimental import pallas as pl
from jax.experimental.pallas import tpu as pltpu
from jax.experimental.pallas import tpu_sc as plsc

NUM_G = 16
L = 16
NC = 2
NS = 16
NW = NC * NS
B = 1024
TC_PER_B = B // 128

_MAGIC = np.int32(0x5F3759DF)


def _rsqrt_newton(s2):
    i = lax.bitcast_convert_type(s2, jnp.int32)
    y = lax.bitcast_convert_type(_MAGIC - (i >> 1), jnp.float32)
    for _ in range(3):
        y = y * (np.float32(1.5) - np.float32(0.5) * s2 * y * y)
    return y


def _compute_block(shared_unused, rows_j, rows_i, obuf, par, ngroups):
    eidx0 = lax.iota(jnp.int32, L)
    c0 = jnp.zeros((L,), jnp.int32)
    c1 = jnp.ones((L,), jnp.int32)
    c2 = jnp.full((L,), 2, jnp.int32)
    shift_v = par[0, :]
    neg_inv = par[1, :]

    def grp(k, carry):
        e0 = k * L
        eidx = e0 + eidx0
        xj = plsc.load_gather(rows_j, [eidx, c0])
        yj = plsc.load_gather(rows_j, [eidx, c1])
        zj = plsc.load_gather(rows_j, [eidx, c2])
        xi = plsc.load_gather(rows_i, [eidx, c0])
        yi = plsc.load_gather(rows_i, [eidx, c1])
        zi = plsc.load_gather(rows_i, [eidx, c2])
        dx = xj - xi
        dy = yj - yi
        dz = zj - zi
        s2 = dx * dx + dy * dy + dz * dz
        n = s2 * _rsqrt_newton(s2)
        n = jnp.where(s2 > np.float32(0.0), n, np.float32(0.0))
        cl = k // (128 // L)
        l0 = (k % (128 // L)) * L
        for g in range(NUM_G):
            t = n - shift_v[g]
            obuf[g // 8, cl, g % 8, pl.ds(l0, L)] = jnp.exp(t * t * neg_inv[g])
        return carry

    lax.fori_loop(0, ngroups, grp, 0, unroll=2)


def _body(npad, pairs_total, rem_tc, pos8, ei, shift, scale, out,
          shared, idx_j0, idx_j1, idx_i0, idx_i1,
          rows_j0, rows_j1, rows_i0, rows_i1, obuf0, obuf1, par,
          sem_s, sem_x0, sem_x1, sem_g0, sem_g1, sem_o0, sem_o1):
    sid = lax.axis_index("s")
    wid = sid * NC + lax.axis_index("c")

    rows_per_tile = npad // NS
    pltpu.sync_copy(pos8.at[pl.ds(sid * rows_per_tile, rows_per_tile)],
                    shared.at[pl.ds(sid * rows_per_tile, rows_per_tile)])
    pltpu.sync_copy(shift, par.at[0])
    pltpu.sync_copy(scale, par.at[1])
    sc = par[1, :]
    par[1, :] = np.float32(-0.5) / (sc * sc)
    plsc.subcore_barrier()

    ppw = pairs_total // NW
    extra = pairs_total % NW
    npair = ppw + jnp.where(wid < extra, 1, 0)
    pair_base = ppw * wid + jnp.minimum(wid, extra)
    nb = 2 * npair

    idx_j = (idx_j0, idx_j1)
    idx_i = (idx_i0, idx_i1)
    rows_j = (rows_j0, rows_j1)
    rows_i = (rows_i0, rows_i1)
    obuf = (obuf0, obuf1)
    sem_x = (sem_x0, sem_x1)
    sem_g = (sem_g0, sem_g1)
    sem_o = (sem_o0, sem_o1)

    def block_off(b):
        return (pair_base * 2 + b) * B

    def issue_idx(b, p):
        off = block_off(b)
        pltpu.async_copy(ei.at[0, pl.ds(off, B)], idx_j[p], sem_x[p])
        pltpu.async_copy(ei.at[1, pl.ds(off, B)], idx_i[p], sem_x[p])

    def wait_idx(b, p):
        pltpu.make_async_copy(ei.at[0, pl.ds(0, B)], idx_j[p], sem_x[p]).wait()
        pltpu.make_async_copy(ei.at[1, pl.ds(0, B)], idx_i[p], sem_x[p]).wait()

    def issue_gather(p):
        pltpu.async_copy(shared.at[idx_j[p]], rows_j[p], sem_g[p])
        pltpu.async_copy(shared.at[idx_i[p]], rows_i[p], sem_g[p])

    def wait_gather(p):
        pltpu.make_async_copy(shared.at[idx_j[p]], rows_j[p], sem_g[p]).wait()
        pltpu.make_async_copy(shared.at[idx_i[p]], rows_i[p], sem_g[p]).wait()

    def out_slice(b):
        return out.at[:, pl.ds((pair_base * 2 + b) * TC_PER_B, TC_PER_B)]

    def issue_out(b, p):
        pltpu.async_copy(obuf[p], out_slice(b), sem_o[p])

    def wait_out(p):
        pltpu.make_async_copy(obuf[p], out_slice(0), sem_o[p]).wait()

    @pl.when(nb >= 1)
    def _():
        issue_idx(0, 0)

    @pl.when(nb >= 2)
    def _():
        issue_idx(1, 1)

    @pl.when(nb >= 1)
    def _():
        wait_idx(0, 0)
        issue_gather(0)

    def pair(p, carry):
        for half in (0, 1):
            b = 2 * p + half
            q = 1 - half
            wait_gather(half)

            @pl.when(b + 2 < nb)
            def _():
                issue_idx(b + 2, half)

            @pl.when(b + 1 < nb)
            def _():
                wait_idx(b + 1, q)
                issue_gather(q)

            @pl.when(b >= 2)
            def _():
                wait_out(half)

            _compute_block(shared, rows_j[half], rows_i[half], obuf[half],
                           par, B // L)
            issue_out(b, half)
        return carry

    lax.fori_loop(0, npair, pair, 0, unroll=False)

    @pl.when(nb >= 2)
    def _():
        wait_out(0)

    @pl.when(nb >= 1)
    def _():
        wait_out(1)

    @pl.when(wid < rem_tc)
    def _():
        tc0 = pairs_total * 2 * TC_PER_B + wid
        off = tc0 * 128
        pltpu.sync_copy(ei.at[0, pl.ds(off, 128)], idx_j0.at[pl.ds(0, 128)])
        pltpu.sync_copy(ei.at[1, pl.ds(off, 128)], idx_i0.at[pl.ds(0, 128)])
        jd = pltpu.async_copy(shared.at[idx_j0.at[pl.ds(0, 128)]],
                              rows_j0.at[pl.ds(0, 128)], sem_g0)
        idd = pltpu.async_copy(shared.at[idx_i0.at[pl.ds(0, 128)]],
                               rows_i0.at[pl.ds(0, 128)], sem_g0)
        jd.wait()
        idd.wait()
        _compute_block(shared, rows_j0, rows_i0, obuf0, par, 128 // L)
        pltpu.sync_copy(obuf0.at[:, pl.ds(0, 1)], out.at[:, pl.ds(tc0, 1)])


def kernel(pos, edge_index, shift, scale):
    n_nodes = pos.shape[0]
    n_edges = edge_index.shape[1]
    ei = edge_index.astype(jnp.int32)
    npad = -(-n_nodes // NS) * NS
    pos8 = jnp.pad(pos.astype(jnp.float32),
                   ((0, npad - n_nodes), (0, 8 - pos.shape[1])))

    e_pad = -(-n_edges // 128) * 128
    if e_pad != n_edges:
        ei = jnp.pad(ei, ((0, 0), (0, e_pad - n_edges)))
    ec = e_pad // 128
    pairs_total = ec // (2 * TC_PER_B)
    rem_tc = ec % (2 * TC_PER_B)
    assert rem_tc < NW

    mesh = plsc.VectorSubcoreMesh(core_axis_name="c", subcore_axis_name="s")
    f = pl.kernel(
        functools.partial(_body, npad, pairs_total, rem_tc),
        out_type=jax.ShapeDtypeStruct((2, ec, 8, 128), jnp.float32),
        mesh=mesh,
        scratch_types=[
            pltpu.VMEM_SHARED((npad, 8), jnp.float32),
            pltpu.VMEM((B,), jnp.int32),
            pltpu.VMEM((B,), jnp.int32),
            pltpu.VMEM((B,), jnp.int32),
            pltpu.VMEM((B,), jnp.int32),
            pltpu.VMEM((B, 8), jnp.float32),
            pltpu.VMEM((B, 8), jnp.float32),
            pltpu.VMEM((B, 8), jnp.float32),
            pltpu.VMEM((B, 8), jnp.float32),
            pltpu.VMEM((2, TC_PER_B, 8, 128), jnp.float32),
            pltpu.VMEM((2, TC_PER_B, 8, 128), jnp.float32),
            pltpu.VMEM((2, NUM_G), jnp.float32),
            pltpu.SemaphoreType.DMA,
            pltpu.SemaphoreType.DMA,
            pltpu.SemaphoreType.DMA,
            pltpu.SemaphoreType.DMA,
            pltpu.SemaphoreType.DMA,
            pltpu.SemaphoreType.DMA,
            pltpu.SemaphoreType.DMA,
        ],
        compiler_params=pltpu.CompilerParams(
            needs_layout_passes=False,
            use_tc_tiling_on_sc=False,
        ),
        name="gaussian_edge_embed_sc",
    )
    out = f(pos8, ei, shift.astype(jnp.float32), scale.astype(jnp.float32))
    res = out.transpose(1, 3, 0, 2).reshape(e_pad, NUM_G)
    if e_pad != n_edges:
        res = res[:n_edges]
    return res


# speedup vs baseline: 9.5382x; 1.0554x over previous
import functools

import numpy as np
import jax
import jax.numpy as jnp
from jax import lax
from jax.experimental import pallas as pl
from jax.experimental.pallas import tpu as pltpu
from jax.experimental.pallas import tpu_sc as plsc

NUM_G = 16
L = 16
NC = 2
NS = 16
NW = NC * NS
B = 1024
TC_PER_B = B // 128

_MAGIC = np.int32(0x5F3759DF)


def _rsqrt_newton(s2):
    i = lax.bitcast_convert_type(s2, jnp.int32)
    y = lax.bitcast_convert_type(_MAGIC - (i >> 1), jnp.float32)
    for _ in range(2):
        y = y * (np.float32(1.5) - np.float32(0.5) * s2 * y * y)
    return y


def _compute_block(shared_unused, rows_j, rows_i, obuf, par, ngroups):
    eidx0 = lax.iota(jnp.int32, L)
    c0 = jnp.zeros((L,), jnp.int32)
    c1 = jnp.ones((L,), jnp.int32)
    c2 = jnp.full((L,), 2, jnp.int32)
    shift_v = par[0, :]
    neg_inv = par[1, :]

    def grp(k, carry):
        e0 = k * L
        eidx = e0 + eidx0
        xj = plsc.load_gather(rows_j, [eidx, c0])
        yj = plsc.load_gather(rows_j, [eidx, c1])
        zj = plsc.load_gather(rows_j, [eidx, c2])
        xi = plsc.load_gather(rows_i, [eidx, c0])
        yi = plsc.load_gather(rows_i, [eidx, c1])
        zi = plsc.load_gather(rows_i, [eidx, c2])
        dx = xj - xi
        dy = yj - yi
        dz = zj - zi
        s2 = dx * dx + dy * dy + dz * dz
        n = s2 * _rsqrt_newton(s2)
        n = jnp.where(s2 > np.float32(0.0), n, np.float32(0.0))
        cl = k // (128 // L)
        l0 = (k % (128 // L)) * L
        for g in range(NUM_G):
            t = n - shift_v[g]
            obuf[g // 8, cl, g % 8, pl.ds(l0, L)] = jnp.exp(t * t * neg_inv[g])
        return carry

    lax.fori_loop(0, ngroups, grp, 0, unroll=4)


def _body(npad, pairs_total, rem_tc, pos8, ei, shift, scale, out,
          shared, idx_j0, idx_j1, idx_i0, idx_i1,
          rows_j0, rows_j1, rows_i0, rows_i1, obuf0, obuf1, par,
          sem_s, sem_x0, sem_x1, sem_g0, sem_g1, sem_o0, sem_o1):
    sid = lax.axis_index("s")
    wid = sid * NC + lax.axis_index("c")

    rows_per_tile = npad // NS
    pltpu.sync_copy(pos8.at[pl.ds(sid * rows_per_tile, rows_per_tile)],
                    shared.at[pl.ds(sid * rows_per_tile, rows_per_tile)])
    pltpu.sync_copy(shift, par.at[0])
    pltpu.sync_copy(scale, par.at[1])
    sc = par[1, :]
    par[1, :] = np.float32(-0.5) / (sc * sc)
    plsc.subcore_barrier()

    ppw = pairs_total // NW
    extra = pairs_total % NW
    npair = ppw + jnp.where(wid < extra, 1, 0)
    pair_base = ppw * wid + jnp.minimum(wid, extra)
    nb = 2 * npair

    idx_j = (idx_j0, idx_j1)
    idx_i = (idx_i0, idx_i1)
    rows_j = (rows_j0, rows_j1)
    rows_i = (rows_i0, rows_i1)
    obuf = (obuf0, obuf1)
    sem_x = (sem_x0, sem_x1)
    sem_g = (sem_g0, sem_g1)
    sem_o = (sem_o0, sem_o1)

    def block_off(b):
        return (pair_base * 2 + b) * B

    def issue_idx(b, p):
        off = block_off(b)
        pltpu.async_copy(ei.at[0, pl.ds(off, B)], idx_j[p], sem_x[p])
        pltpu.async_copy(ei.at[1, pl.ds(off, B)], idx_i[p], sem_x[p])

    def wait_idx(b, p):
        pltpu.make_async_copy(ei.at[0, pl.ds(0, B)], idx_j[p], sem_x[p]).wait()
        pltpu.make_async_copy(ei.at[1, pl.ds(0, B)], idx_i[p], sem_x[p]).wait()

    def issue_gather(p):
        pltpu.async_copy(shared.at[idx_j[p]], rows_j[p], sem_g[p])
        pltpu.async_copy(shared.at[idx_i[p]], rows_i[p], sem_g[p])

    def wait_gather(p):
        pltpu.make_async_copy(shared.at[idx_j[p]], rows_j[p], sem_g[p]).wait()
        pltpu.make_async_copy(shared.at[idx_i[p]], rows_i[p], sem_g[p]).wait()

    def out_slice(b):
        return out.at[:, pl.ds((pair_base * 2 + b) * TC_PER_B, TC_PER_B)]

    def issue_out(b, p):
        pltpu.async_copy(obuf[p], out_slice(b), sem_o[p])

    def wait_out(p):
        pltpu.make_async_copy(obuf[p], out_slice(0), sem_o[p]).wait()

    @pl.when(nb >= 1)
    def _():
        issue_idx(0, 0)

    @pl.when(nb >= 2)
    def _():
        issue_idx(1, 1)

    @pl.when(nb >= 1)
    def _():
        wait_idx(0, 0)
        issue_gather(0)

    def pair(p, carry):
        for half in (0, 1):
            b = 2 * p + half
            q = 1 - half
            wait_gather(half)

            @pl.when(b + 2 < nb)
            def _():
                issue_idx(b + 2, half)

            @pl.when(b + 1 < nb)
            def _():
                wait_idx(b + 1, q)
                issue_gather(q)

            @pl.when(b >= 2)
            def _():
                wait_out(half)

            _compute_block(shared, rows_j[half], rows_i[half], obuf[half],
                           par, B // L)
            issue_out(b, half)
        return carry

    lax.fori_loop(0, npair, pair, 0, unroll=False)

    @pl.when(nb >= 2)
    def _():
        wait_out(0)

    @pl.when(nb >= 1)
    def _():
        wait_out(1)

    @pl.when(wid < rem_tc)
    def _():
        tc0 = pairs_total * 2 * TC_PER_B + wid
        off = tc0 * 128
        pltpu.sync_copy(ei.at[0, pl.ds(off, 128)], idx_j0.at[pl.ds(0, 128)])
        pltpu.sync_copy(ei.at[1, pl.ds(off, 128)], idx_i0.at[pl.ds(0, 128)])
        jd = pltpu.async_copy(shared.at[idx_j0.at[pl.ds(0, 128)]],
                              rows_j0.at[pl.ds(0, 128)], sem_g0)
        idd = pltpu.async_copy(shared.at[idx_i0.at[pl.ds(0, 128)]],
                               rows_i0.at[pl.ds(0, 128)], sem_g0)
        jd.wait()
        idd.wait()
        _compute_block(shared, rows_j0, rows_i0, obuf0, par, 128 // L)
        pltpu.sync_copy(obuf0.at[:, pl.ds(0, 1)], out.at[:, pl.ds(tc0, 1)])


def kernel(pos, edge_index, shift, scale):
    n_nodes = pos.shape[0]
    n_edges = edge_index.shape[1]
    ei = edge_index.astype(jnp.int32)
    npad = -(-n_nodes // NS) * NS
    pos8 = jnp.pad(pos.astype(jnp.float32),
                   ((0, npad - n_nodes), (0, 8 - pos.shape[1])))

    e_pad = -(-n_edges // 128) * 128
    if e_pad != n_edges:
        ei = jnp.pad(ei, ((0, 0), (0, e_pad - n_edges)))
    ec = e_pad // 128
    pairs_total = ec // (2 * TC_PER_B)
    rem_tc = ec % (2 * TC_PER_B)
    assert rem_tc < NW

    mesh = plsc.VectorSubcoreMesh(core_axis_name="c", subcore_axis_name="s")
    f = pl.kernel(
        functools.partial(_body, npad, pairs_total, rem_tc),
        out_type=jax.ShapeDtypeStruct((2, ec, 8, 128), jnp.float32),
        mesh=mesh,
        scratch_types=[
            pltpu.VMEM_SHARED((npad, 8), jnp.float32),
            pltpu.VMEM((B,), jnp.int32),
            pltpu.VMEM((B,), jnp.int32),
            pltpu.VMEM((B,), jnp.int32),
            pltpu.VMEM((B,), jnp.int32),
            pltpu.VMEM((B, 8), jnp.float32),
            pltpu.VMEM((B, 8), jnp.float32),
            pltpu.VMEM((B, 8), jnp.float32),
            pltpu.VMEM((B, 8), jnp.float32),
            pltpu.VMEM((2, TC_PER_B, 8, 128), jnp.float32),
            pltpu.VMEM((2, TC_PER_B, 8, 128), jnp.float32),
            pltpu.VMEM((2, NUM_G), jnp.float32),
            pltpu.SemaphoreType.DMA,
            pltpu.SemaphoreType.DMA,
            pltpu.SemaphoreType.DMA,
            pltpu.SemaphoreType.DMA,
            pltpu.SemaphoreType.DMA,
            pltpu.SemaphoreType.DMA,
            pltpu.SemaphoreType.DMA,
        ],
        compiler_params=pltpu.CompilerParams(
            needs_layout_passes=False,
            use_tc_tiling_on_sc=False,
        ),
        name="gaussian_edge_embed_sc",
    )
    out = f(pos8, ei, shift.astype(jnp.float32), scale.astype(jnp.float32))
    res = out.transpose(1, 3, 0, 2).reshape(e_pad, NUM_G)
    if e_pad != n_edges:
        res = res[:n_edges]
    return res
